# baseline (speedup 1.0000x reference)
import functools

import jax
import jax.numpy as jnp
from jax import lax
from jax.experimental import pallas as pl
from jax.experimental.pallas import tpu as pltpu
from jax.experimental.pallas import tpu_sc as plsc

N = 10000
EMB = 256
EXT = 288
NE = 160000
RB = 1000
GRID = N // RB

HALF = EXT // 2
VPR = HALF // 16
NSUB = 16
NPAD = 10240
RPT = NPAD // NSUB
EK = 48
EPT = 10176
NCH = EPT // EK
NE_P = NSUB * EPT
NCHG = NE_P // EK

_f32 = jnp.float32


def _dot(a, b):
    return jnp.dot(a, b, preferred_element_type=_f32)


def _embed_body(x_ref, w1_ref, b1_ref, w2_ref, b2_ref, emb_ref):
    h = jnp.maximum(_dot(x_ref[...], w1_ref[...]) + b1_ref[...], 0.0)
    emb_ref[...] = jnp.maximum(_dot(h, w2_ref[...]) + b2_ref[...], 0.0)


def _embed(x, w1, b1, w2, b2):
    full = lambda shape: pl.BlockSpec(shape, lambda i: (0, 0))
    return pl.pallas_call(
        _embed_body,
        grid=(GRID,),
        in_specs=[pl.BlockSpec((RB, EMB), lambda i: (i, 0)),
                  full((EMB, EMB)), full((1, EMB)), full((EMB, EMB)), full((1, EMB))],
        out_specs=pl.BlockSpec((RB, EMB), lambda i: (i, 0)),
        out_shape=jax.ShapeDtypeStruct((N, EMB), _f32),
    )(x, w1, b1.reshape(1, EMB), w2, b2.reshape(1, EMB))


def _tables_body(nt, x_ref, *refs):
    x = x_ref[...]
    for t in range(nt):
        w_ref, b_ref, o_ref = refs[t], refs[nt + t], refs[2 * nt + t]
        o_ref[...] = _dot(x, w_ref[0]) + b_ref[0]


def _split_tables(x, wbs):
    nt = len(wbs)
    ws = [w.reshape(EMB, 2, HALF).transpose(1, 0, 2) for w, _ in wbs]
    bs = [b.reshape(2, 1, HALF) for _, b in wbs]
    full3 = lambda shape: pl.BlockSpec(shape, lambda i, c: (c, 0, 0))
    outs = pl.pallas_call(
        functools.partial(_tables_body, nt),
        grid=(GRID, 2),
        in_specs=([pl.BlockSpec((RB, EMB), lambda i, c: (i, 0))]
                  + [full3((1, EMB, HALF))] * nt + [full3((1, 1, HALF))] * nt),
        out_specs=[pl.BlockSpec((RB, HALF), lambda i, c: (c * GRID + i, 0))] * nt,
        out_shape=[jax.ShapeDtypeStruct((2 * N, HALF), _f32)] * nt,
    )(x, *ws, *bs)
    return list(outs)


def _escal_body(x_ref, wb_ref, o_ref):
    o_ref[...] = jnp.maximum(x_ref[...] * wb_ref[0, 0] + wb_ref[0, 1], 0.0)


def _edge_scalars(ef, ew, eb):
    x = ef.reshape(NE // 128, 128)
    wb = jnp.stack([ew.reshape(()), eb.reshape(())]).reshape(1, 2)
    o = pl.pallas_call(
        _escal_body,
        in_specs=[pl.BlockSpec(x.shape, lambda: (0, 0)),
                  pl.BlockSpec((1, 2), lambda: (0, 0))],
        out_specs=pl.BlockSpec(x.shape, lambda: (0, 0)),
        out_shape=jax.ShapeDtypeStruct(x.shape, _f32),
    )(x, wb)
    return o.reshape(NE)


def _post_body(want_y, s0_ref, s1_ref, r_ref, fw_ref, fb_ref, pw_ref, pb_ref,
               o1a_ref, o1b_ref, o1bias_ref, o2w_ref, o2b_ref,
               zw_ref, zb_ref, *out_refs):
    s0 = s0_ref[0]
    s1 = s1_ref[0]
    fw = fw_ref[...]
    nfb = EMB - HALF
    agg = (_dot(s0, fw[:HALF]) + _dot(s1[:, :nfb], fw[HALF:])
           + s1[:, nfb:nfb + 1] * fb_ref[...])
    post = _dot(jnp.maximum(agg, 0.0), pw_ref[...]) + pb_ref[...]
    h = jnp.maximum(_dot(post, o1a_ref[...]) + _dot(r_ref[...], o1b_ref[...])
                    + o1bias_ref[...], 0.0)
    y = _dot(h, o2w_ref[...]) + o2b_ref[...]
    out_refs[0][...] = jnp.maximum(_dot(y, zw_ref[...]) + zb_ref[...], 0.0)
    if want_y:
        out_refs[1][...] = y


def _post(s_raw, right, p, zw, zb, want_y):
    (lw, lb, ew, rw, fw, fb, pw, pb, o1w, o1b, o2w, o2b) = p
    full = lambda shape: pl.BlockSpec(shape, lambda i: (0, 0))
    in_specs = [pl.BlockSpec((1, RB, HALF), lambda i: (0, i, 0)),
                pl.BlockSpec((1, RB, HALF), lambda i: (1, i, 0)),
                pl.BlockSpec((RB, EMB), lambda i: (i, 0)),
                full((EMB, EMB)), full((1, EMB)),
                full((EMB, EMB)), full((1, EMB)),
                full((EMB, EMB)), full((EMB, EMB)), full((1, EMB)),
                full((EMB, EMB)), full((1, EMB)),
                full((EMB, EMB)), full((1, EMB))]
    nout = 2 if want_y else 1
    outs = pl.pallas_call(
        functools.partial(_post_body, want_y),
        grid=(GRID,), in_specs=in_specs,
        out_specs=[pl.BlockSpec((RB, EMB), lambda i: (i, 0))] * nout,
        out_shape=[jax.ShapeDtypeStruct((N, EMB), _f32)] * nout,
    )(s_raw, s_raw, right,
      fw, fb.reshape(1, EMB), pw, pb.reshape(1, EMB),
      o1w[:EMB], o1w[EMB:], o1b.reshape(1, EMB),
      o2w, o2b.reshape(1, EMB), zw, zb.reshape(1, EMB))
    return outs if want_y else (outs[0],)


def _ext_w(w):
    return jnp.pad(w, ((0, 0), (0, EXT - EMB)))


def _ext_b(b_or_none):
    b = jnp.zeros((EMB,), _f32) if b_or_none is None else b_or_none
    return jnp.concatenate([b, jnp.full((1,), 0.5, _f32),
                            jnp.zeros((EXT - EMB - 1,), _f32)])


def _ext_ew(ew):
    return jnp.concatenate([ew.reshape(EMB), jnp.zeros((EXT - EMB,), _f32)])


def _pad_edges(idx):
    pad = N + (jnp.arange(NE_P - NE, dtype=jnp.int32) % (NPAD - N))
    return jnp.concatenate([idx.astype(jnp.int32), pad])


def _edge_sc_body(a_hbm, b_hbm, pk_hbm, ew_hbm, out_hbm,
                  ia0, ia1, ib0, ib1, io0, io1, pk0, pk1,
                  ab0, ab1, bb0, bb1, ewv, s_sp, sa0, sa1, sb0, sb1):
    cid = lax.axis_index("c")
    sid = lax.axis_index("s")
    cofs = cid * N

    pltpu.sync_copy(ew_hbm.at[pl.ds(cid * HALF, HALF)], ewv)

    def zrow(k, c):
        for r in range(VPR):
            ab0[k, pl.ds(r * 16, 16)] = jnp.zeros((16,), _f32)
        return c
    lax.fori_loop(0, EK, zrow, 0)
    rbase = sid * RPT
    for t in range(RPT // EK):
        pltpu.sync_copy(ab0, s_sp.at[pl.ds(rbase + t * EK, EK), :])
    rem = RPT % EK
    if rem:
        pltpu.sync_copy(ab0.at[pl.ds(0, rem), :],
                        s_sp.at[pl.ds(rbase + RPT - rem, rem), :])
    plsc.subcore_barrier()

    slots = ((ia0, ib0, io0, pk0, ab0, bb0, sa0, sb0),
             (ia1, ib1, io1, pk1, ab1, bb1, sa1, sb1))

    def fire(j, slot):
        ia, ib, io, pk, ab, bb, sa, sb = slot
        row = sid * NCH + j
        pltpu.sync_copy(pk_hbm.at[row], pk)
        for q in range(EK // 16):
            sl = pl.ds(q * 16, 16)
            sv = plsc.bitcast(pk[sl], jnp.int32)
            dv = plsc.bitcast(pk[pl.ds(EK + q * 16, 16)], jnp.int32)
            ia[sl] = jnp.minimum(sv, N - 1) + cofs
            ib[sl] = dv
            io[sl] = jnp.minimum(dv, N - 1) + cofs
        return (pltpu.async_copy(a_hbm.at[ia], ab, sa),
                pltpu.async_copy(b_hbm.at[io], bb, sb))

    def consume(slot, descs):
        ia, ib, io, pk, ab, bb, sa, sb = slot
        descs[0].wait()
        descs[1].wait()

        @plsc.parallel_loop(0, EK, 1, unroll=4)
        def edge(k):
            ev = pk[pl.ds(2 * EK + k * 16, 16)]
            for r in range(VPR):
                sl = pl.ds(r * 16, 16)
                ab[k, sl] = jnp.maximum(
                    ab[k, sl] + bb[k, sl] + ev * ewv[sl], 0.0)
        pltpu.sync_copy(ab, s_sp.at[ib], add=True)

    def pair(jj, c):
        j0 = 2 * jj
        d0 = fire(j0, slots[0])
        d1 = fire(j0 + 1, slots[1])
        consume(slots[0], d0)
        consume(slots[1], d1)
        return c
    lax.fori_loop(0, NCH // 2, pair, 0)

    plsc.subcore_barrier()
    pltpu.sync_copy(s_sp.at[pl.ds(rbase, RPT), :],
                    out_hbm.at[cid, pl.ds(rbase, RPT), :])


def _edge_stage(a_tab, b_tab, pk, ew_ext):
    fn = pl.kernel(
        _edge_sc_body,
        out_type=jax.ShapeDtypeStruct((2, NPAD, HALF), _f32),
        mesh=plsc.VectorSubcoreMesh(core_axis_name="c", subcore_axis_name="s",
                                    num_cores=2, num_subcores=NSUB),
        scratch_types=(
            [pltpu.VMEM((EK,), jnp.int32)] * 6
            + [pltpu.VMEM((18 * EK,), _f32)] * 2
            + [pltpu.VMEM((EK, HALF), _f32)] * 4
            + [pltpu.VMEM((HALF,), _f32),
               pltpu.VMEM_SHARED((NPAD, HALF), _f32)]
            + [pltpu.SemaphoreType.DMA] * 4
        ),
        compiler_params=pltpu.CompilerParams(use_tc_tiling_on_sc=False,
                                             needs_layout_passes=False),
    )
    return fn(a_tab, b_tab, pk, ew_ext)


def kernel(constraint_features, edge_indices, edge_features, variable_features,
           ce, ee, ve, vc, cv, ov, oc):
    (vc_lw, vc_lb, vc_ew, vc_rw, vc_fw, vc_fb, vc_pw, vc_pb,
     vc_o1w, vc_o1b, vc_o2w, vc_o2b) = vc
    (cv_lw, cv_lb, cv_ew, cv_rw, cv_fw, cv_fb, cv_pw, cv_pb,
     cv_o1w, cv_o1b, cv_o2w, cv_o2b) = cv

    src_c = _pad_edges(edge_indices[0])
    src_v = _pad_edges(edge_indices[1])

    c1 = _embed(constraint_features, ce[0], ce[1], ce[2], ce[3])
    v1 = _embed(variable_features, ve[0], ve[1], ve[2], ve[3])
    (tb1,) = _split_tables(c1, [(_ext_w(vc_rw), _ext_b(None))])
    ta1, tb2 = _split_tables(v1, [(_ext_w(vc_lw), _ext_b(vc_lb)),
                                  (_ext_w(cv_rw), _ext_b(None))])
    esc = _edge_scalars(edge_features, ee[0], ee[1])
    esc_p = jnp.broadcast_to(
        jnp.concatenate([esc, jnp.zeros((NE_P - NE,), _f32)])[:, None],
        (NE_P, 16)).reshape(NCHG, EK * 16)

    def _packed(gather_idx, scatter_idx):
        as_f32 = lambda a: jax.lax.bitcast_convert_type(
            a.reshape(NCHG, EK), _f32)
        return jnp.concatenate(
            [as_f32(gather_idx), as_f32(scatter_idx), esc_p], axis=1)

    s1 = _edge_stage(ta1, tb1, _packed(src_v, src_c), _ext_ew(vc_ew))
    z_c, c2 = _post(s1, c1, vc, oc[0], oc[1], want_y=True)
    (ta2,) = _split_tables(c2, [(_ext_w(cv_lw), _ext_b(cv_lb))])

    s2 = _edge_stage(ta2, tb2, _packed(src_c, src_v), _ext_ew(cv_ew))
    (z_v,) = _post(s2, v1, cv, ov[0], ov[1], want_y=False)

    return (z_v, z_c)


# speedup vs baseline: 7.1187x; 7.1187x over previous
import functools

import jax
import jax.numpy as jnp
from jax import lax
from jax.experimental import pallas as pl
from jax.experimental.pallas import tpu as pltpu
from jax.experimental.pallas import tpu_sc as plsc

N = 10000
EMB = 256
EXT = 288
NE = 160000
RB = 1000
GRID = N // RB

HALF = EXT // 2
VPR = HALF // 16
NSUB = 16
NPAD = 10240
RPT = NPAD // NSUB
EK = 48
EPT = 10176
NCH = EPT // EK
NE_P = NSUB * EPT

_f32 = jnp.float32


def _dot(a, b):
    return jnp.dot(a, b, preferred_element_type=_f32)


def _embed_body(x_ref, w1_ref, b1_ref, w2_ref, b2_ref, emb_ref):
    h = jnp.maximum(_dot(x_ref[...], w1_ref[...]) + b1_ref[...], 0.0)
    emb_ref[...] = jnp.maximum(_dot(h, w2_ref[...]) + b2_ref[...], 0.0)


def _embed(x, w1, b1, w2, b2):
    full = lambda shape: pl.BlockSpec(shape, lambda i: (0, 0))
    return pl.pallas_call(
        _embed_body,
        grid=(GRID,),
        in_specs=[pl.BlockSpec((RB, EMB), lambda i: (i, 0)),
                  full((EMB, EMB)), full((1, EMB)), full((EMB, EMB)), full((1, EMB))],
        out_specs=pl.BlockSpec((RB, EMB), lambda i: (i, 0)),
        out_shape=jax.ShapeDtypeStruct((N, EMB), _f32),
    )(x, w1, b1.reshape(1, EMB), w2, b2.reshape(1, EMB))


def _tables_body(nt, x_ref, *refs):
    x = x_ref[...]
    for t in range(nt):
        w_ref, b_ref, o_ref = refs[t], refs[nt + t], refs[2 * nt + t]
        o_ref[...] = _dot(x, w_ref[0]) + b_ref[0]


def _split_tables(x, wbs):
    nt = len(wbs)
    ws = [w.reshape(EMB, 2, HALF).transpose(1, 0, 2) for w, _ in wbs]
    bs = [b.reshape(2, 1, HALF) for _, b in wbs]
    full3 = lambda shape: pl.BlockSpec(shape, lambda i, c: (c, 0, 0))
    outs = pl.pallas_call(
        functools.partial(_tables_body, nt),
        grid=(GRID, 2),
        in_specs=([pl.BlockSpec((RB, EMB), lambda i, c: (i, 0))]
                  + [full3((1, EMB, HALF))] * nt + [full3((1, 1, HALF))] * nt),
        out_specs=[pl.BlockSpec((RB, HALF), lambda i, c: (c * GRID + i, 0))] * nt,
        out_shape=[jax.ShapeDtypeStruct((2 * N, HALF), _f32)] * nt,
    )(x, *ws, *bs)
    return list(outs)


def _escal_body(x_ref, wb_ref, o_ref):
    o_ref[...] = jnp.maximum(x_ref[...] * wb_ref[0, 0] + wb_ref[0, 1], 0.0)


def _edge_scalars(ef, ew, eb):
    x = ef.reshape(NE // 128, 128)
    wb = jnp.stack([ew.reshape(()), eb.reshape(())]).reshape(1, 2)
    o = pl.pallas_call(
        _escal_body,
        in_specs=[pl.BlockSpec(x.shape, lambda: (0, 0)),
                  pl.BlockSpec((1, 2), lambda: (0, 0))],
        out_specs=pl.BlockSpec(x.shape, lambda: (0, 0)),
        out_shape=jax.ShapeDtypeStruct(x.shape, _f32),
    )(x, wb)
    return o.reshape(NE)


def _post_body(want_y, s0_ref, s1_ref, r_ref, fw_ref, fb_ref, pw_ref, pb_ref,
               o1a_ref, o1b_ref, o1bias_ref, o2w_ref, o2b_ref,
               zw_ref, zb_ref, *out_refs):
    s0 = s0_ref[0]
    s1 = s1_ref[0]
    fw = fw_ref[...]
    nfb = EMB - HALF
    agg = (_dot(s0, fw[:HALF]) + _dot(s1[:, :nfb], fw[HALF:])
           + s1[:, nfb:nfb + 1] * fb_ref[...])
    post = _dot(jnp.maximum(agg, 0.0), pw_ref[...]) + pb_ref[...]
    h = jnp.maximum(_dot(post, o1a_ref[...]) + _dot(r_ref[...], o1b_ref[...])
                    + o1bias_ref[...], 0.0)
    y = _dot(h, o2w_ref[...]) + o2b_ref[...]
    out_refs[0][...] = jnp.maximum(_dot(y, zw_ref[...]) + zb_ref[...], 0.0)
    if want_y:
        out_refs[1][...] = y


def _post(s_raw, right, p, zw, zb, want_y):
    (lw, lb, ew, rw, fw, fb, pw, pb, o1w, o1b, o2w, o2b) = p
    full = lambda shape: pl.BlockSpec(shape, lambda i: (0, 0))
    in_specs = [pl.BlockSpec((1, RB, HALF), lambda i: (0, i, 0)),
                pl.BlockSpec((1, RB, HALF), lambda i: (1, i, 0)),
                pl.BlockSpec((RB, EMB), lambda i: (i, 0)),
                full((EMB, EMB)), full((1, EMB)),
                full((EMB, EMB)), full((1, EMB)),
                full((EMB, EMB)), full((EMB, EMB)), full((1, EMB)),
                full((EMB, EMB)), full((1, EMB)),
                full((EMB, EMB)), full((1, EMB))]
    nout = 2 if want_y else 1
    outs = pl.pallas_call(
        functools.partial(_post_body, want_y),
        grid=(GRID,), in_specs=in_specs,
        out_specs=[pl.BlockSpec((RB, EMB), lambda i: (i, 0))] * nout,
        out_shape=[jax.ShapeDtypeStruct((N, EMB), _f32)] * nout,
    )(s_raw, s_raw, right,
      fw, fb.reshape(1, EMB), pw, pb.reshape(1, EMB),
      o1w[:EMB], o1w[EMB:], o1b.reshape(1, EMB),
      o2w, o2b.reshape(1, EMB), zw, zb.reshape(1, EMB))
    return outs if want_y else (outs[0],)


def _ext_w(w):
    return jnp.pad(w, ((0, 0), (0, EXT - EMB)))


def _ext_b(b_or_none):
    b = jnp.zeros((EMB,), _f32) if b_or_none is None else b_or_none
    return jnp.concatenate([b, jnp.full((1,), 0.5, _f32),
                            jnp.zeros((EXT - EMB - 1,), _f32)])


def _ext_ew(ew):
    return jnp.concatenate([ew.reshape(EMB), jnp.zeros((EXT - EMB,), _f32)])


def _pad_edges(idx):
    pad = N + (jnp.arange(NE_P - NE, dtype=jnp.int32) % (NPAD - N))
    return jnp.concatenate([idx.astype(jnp.int32), pad])


def _edge_sc_body(a_hbm, b_hbm, src_hbm, dst_hbm, esc_hbm, ew_hbm, out_hbm,
                  ia0, ia1, ib0, ib1, io0, io1, es0, es1,
                  ab0, ab1, bb0, bb1, ewv, s_sp, sa0, sa1, sb0, sb1):
    cid = lax.axis_index("c")
    sid = lax.axis_index("s")
    cofs = cid * N

    pltpu.sync_copy(ew_hbm.at[pl.ds(cid * HALF, HALF)], ewv)

    def zrow(k, c):
        for r in range(VPR):
            ab0[k, pl.ds(r * 16, 16)] = jnp.zeros((16,), _f32)
        return c
    lax.fori_loop(0, EK, zrow, 0)
    rbase = sid * RPT
    for t in range(RPT // EK):
        pltpu.sync_copy(ab0, s_sp.at[pl.ds(rbase + t * EK, EK), :])
    rem = RPT % EK
    if rem:
        pltpu.sync_copy(ab0.at[pl.ds(0, rem), :],
                        s_sp.at[pl.ds(rbase + RPT - rem, rem), :])
    plsc.subcore_barrier()

    ebase = sid * EPT
    slots = ((ia0, ib0, io0, es0, ab0, bb0, sa0, sb0),
             (ia1, ib1, io1, es1, ab1, bb1, sa1, sb1))

    def fire(j, slot):
        ia, ib, io, es, ab, bb, sa, sb = slot
        base = ebase + j * EK
        pltpu.sync_copy(src_hbm.at[pl.ds(base, EK)], ia)
        pltpu.sync_copy(dst_hbm.at[pl.ds(base, EK)], ib)
        pltpu.sync_copy(esc_hbm.at[pl.ds(base, EK), :], es)
        for q in range(EK // 16):
            sl = pl.ds(q * 16, 16)
            ia[sl] = jnp.minimum(ia[sl], N - 1) + cofs
            io[sl] = jnp.minimum(ib[sl], N - 1) + cofs
        return (pltpu.async_copy(a_hbm.at[ia], ab, sa),
                pltpu.async_copy(b_hbm.at[io], bb, sb))

    def consume(slot, descs):
        ia, ib, io, es, ab, bb, sa, sb = slot
        descs[0].wait()
        descs[1].wait()

        @plsc.parallel_loop(0, EK, 1, unroll=4)
        def edge(k):
            ev = es[k, :]
            for r in range(VPR):
                sl = pl.ds(r * 16, 16)
                ab[k, sl] = jnp.maximum(
                    ab[k, sl] + bb[k, sl] + ev * ewv[sl], 0.0)
        pltpu.sync_copy(ab, s_sp.at[ib], add=True)

    def pair(jj, c):
        j0 = 2 * jj
        d0 = fire(j0, slots[0])
        d1 = fire(j0 + 1, slots[1])
        consume(slots[0], d0)
        consume(slots[1], d1)
        return c
    lax.fori_loop(0, NCH // 2, pair, 0)

    plsc.subcore_barrier()
    pltpu.sync_copy(s_sp.at[pl.ds(rbase, RPT), :],
                    out_hbm.at[cid, pl.ds(rbase, RPT), :])


def _edge_stage(a_tab, b_tab, src_p, dst_p, esc_p, ew_ext):
    fn = pl.kernel(
        _edge_sc_body,
        out_type=jax.ShapeDtypeStruct((2, NPAD, HALF), _f32),
        mesh=plsc.VectorSubcoreMesh(core_axis_name="c", subcore_axis_name="s",
                                    num_cores=2, num_subcores=NSUB),
        scratch_types=(
            [pltpu.VMEM((EK,), jnp.int32)] * 6
            + [pltpu.VMEM((EK, 16), _f32)] * 2
            + [pltpu.VMEM((EK, HALF), _f32)] * 4
            + [pltpu.VMEM((HALF,), _f32),
               pltpu.VMEM_SHARED((NPAD, HALF), _f32)]
            + [pltpu.SemaphoreType.DMA] * 4
        ),
        compiler_params=pltpu.CompilerParams(use_tc_tiling_on_sc=False),
    )
    return fn(a_tab, b_tab, src_p, dst_p, esc_p, ew_ext)


def kernel(constraint_features, edge_indices, edge_features, variable_features,
           ce, ee, ve, vc, cv, ov, oc):
    (vc_lw, vc_lb, vc_ew, vc_rw, vc_fw, vc_fb, vc_pw, vc_pb,
     vc_o1w, vc_o1b, vc_o2w, vc_o2b) = vc
    (cv_lw, cv_lb, cv_ew, cv_rw, cv_fw, cv_fb, cv_pw, cv_pb,
     cv_o1w, cv_o1b, cv_o2w, cv_o2b) = cv

    src_c = _pad_edges(edge_indices[0])
    src_v = _pad_edges(edge_indices[1])

    c1 = _embed(constraint_features, ce[0], ce[1], ce[2], ce[3])
    v1 = _embed(variable_features, ve[0], ve[1], ve[2], ve[3])
    (tb1,) = _split_tables(c1, [(_ext_w(vc_rw), _ext_b(None))])
    ta1, tb2 = _split_tables(v1, [(_ext_w(vc_lw), _ext_b(vc_lb)),
                                  (_ext_w(cv_rw), _ext_b(None))])
    esc = _edge_scalars(edge_features, ee[0], ee[1])
    esc_p = jnp.broadcast_to(
        jnp.concatenate([esc, jnp.zeros((NE_P - NE,), _f32)])[:, None],
        (NE_P, 16))

    s1 = _edge_stage(ta1, tb1, src_v, src_c, esc_p, _ext_ew(vc_ew))
    z_c, c2 = _post(s1, c1, vc, oc[0], oc[1], want_y=True)
    (ta2,) = _split_tables(c2, [(_ext_w(cv_lw), _ext_b(cv_lb))])

    s2 = _edge_stage(ta2, tb2, src_c, src_v, esc_p, _ext_ew(cv_ew))
    (z_v,) = _post(s2, v1, cv, ov[0], ov[1], want_y=False)

    return (z_v, z_c)


# speedup vs baseline: 7.6997x; 1.0816x over previous
import functools

import jax
import jax.numpy as jnp
from jax import lax
from jax.experimental import pallas as pl
from jax.experimental.pallas import tpu as pltpu
from jax.experimental.pallas import tpu_sc as plsc

N = 10000
EMB = 256
EXT = 288
NE = 160000
RB = 1000
GRID = N // RB

HALF = EXT // 2
VPR = HALF // 16
NSUB = 16
NPAD = 10240
RPT = NPAD // NSUB
EK = 48
EPT = 10176
NCH = EPT // EK
NE_P = NSUB * EPT

_f32 = jnp.float32


def _dot(a, b):
    return jnp.dot(a, b, preferred_element_type=_f32)


def _embed_body(x_ref, w1_ref, b1_ref, w2_ref, b2_ref, emb_ref):
    h = jnp.maximum(_dot(x_ref[...], w1_ref[...]) + b1_ref[...], 0.0)
    emb_ref[...] = jnp.maximum(_dot(h, w2_ref[...]) + b2_ref[...], 0.0)


def _embed(x, w1, b1, w2, b2):
    full = lambda shape: pl.BlockSpec(shape, lambda i: (0, 0))
    return pl.pallas_call(
        _embed_body,
        grid=(GRID,),
        in_specs=[pl.BlockSpec((RB, EMB), lambda i: (i, 0)),
                  full((EMB, EMB)), full((1, EMB)), full((EMB, EMB)), full((1, EMB))],
        out_specs=pl.BlockSpec((RB, EMB), lambda i: (i, 0)),
        out_shape=jax.ShapeDtypeStruct((N, EMB), _f32),
    )(x, w1, b1.reshape(1, EMB), w2, b2.reshape(1, EMB))


def _tables_body(nt, x_ref, *refs):
    x = x_ref[...]
    for t in range(nt):
        w_ref, b_ref, o_ref = refs[t], refs[nt + t], refs[2 * nt + t]
        o_ref[...] = _dot(x, w_ref[0]) + b_ref[0]


def _split_tables(x, wbs):
    nt = len(wbs)
    ws = [w.reshape(EMB, 2, HALF).transpose(1, 0, 2) for w, _ in wbs]
    bs = [b.reshape(2, 1, HALF) for _, b in wbs]
    full3 = lambda shape: pl.BlockSpec(shape, lambda i, c: (c, 0, 0))
    outs = pl.pallas_call(
        functools.partial(_tables_body, nt),
        grid=(GRID, 2),
        in_specs=([pl.BlockSpec((RB, EMB), lambda i, c: (i, 0))]
                  + [full3((1, EMB, HALF))] * nt + [full3((1, 1, HALF))] * nt),
        out_specs=[pl.BlockSpec((RB, HALF), lambda i, c: (c * GRID + i, 0))] * nt,
        out_shape=[jax.ShapeDtypeStruct((2 * N, HALF), _f32)] * nt,
    )(x, *ws, *bs)
    return list(outs)


def _escal_body(x_ref, wb_ref, o_ref):
    o_ref[...] = jnp.maximum(x_ref[...] * wb_ref[0, 0] + wb_ref[0, 1], 0.0)


def _edge_scalars(ef, ew, eb):
    x = ef.reshape(NE // 128, 128)
    wb = jnp.stack([ew.reshape(()), eb.reshape(())]).reshape(1, 2)
    o = pl.pallas_call(
        _escal_body,
        in_specs=[pl.BlockSpec(x.shape, lambda: (0, 0)),
                  pl.BlockSpec((1, 2), lambda: (0, 0))],
        out_specs=pl.BlockSpec(x.shape, lambda: (0, 0)),
        out_shape=jax.ShapeDtypeStruct(x.shape, _f32),
    )(x, wb)
    return o.reshape(NE)


def _post_body(want_y, s0_ref, s1_ref, r_ref, fw_ref, fb_ref, pw_ref, pb_ref,
               o1a_ref, o1b_ref, o1bias_ref, o2w_ref, o2b_ref,
               zw_ref, zb_ref, *out_refs):
    s0 = s0_ref[0]
    s1 = s1_ref[0]
    fw = fw_ref[...]
    nfb = EMB - HALF
    agg = (_dot(s0, fw[:HALF]) + _dot(s1[:, :nfb], fw[HALF:])
           + s1[:, nfb:nfb + 1] * fb_ref[...])
    post = _dot(jnp.maximum(agg, 0.0), pw_ref[...]) + pb_ref[...]
    h = jnp.maximum(_dot(post, o1a_ref[...]) + _dot(r_ref[...], o1b_ref[...])
                    + o1bias_ref[...], 0.0)
    y = _dot(h, o2w_ref[...]) + o2b_ref[...]
    out_refs[0][...] = jnp.maximum(_dot(y, zw_ref[...]) + zb_ref[...], 0.0)
    if want_y:
        out_refs[1][...] = y


def _post(s_raw, right, p, zw, zb, want_y):
    (lw, lb, ew, rw, fw, fb, pw, pb, o1w, o1b, o2w, o2b) = p
    full = lambda shape: pl.BlockSpec(shape, lambda i: (0, 0))
    in_specs = [pl.BlockSpec((1, RB, HALF), lambda i: (0, i, 0)),
                pl.BlockSpec((1, RB, HALF), lambda i: (1, i, 0)),
                pl.BlockSpec((RB, EMB), lambda i: (i, 0)),
                full((EMB, EMB)), full((1, EMB)),
                full((EMB, EMB)), full((1, EMB)),
                full((EMB, EMB)), full((EMB, EMB)), full((1, EMB)),
                full((EMB, EMB)), full((1, EMB)),
                full((EMB, EMB)), full((1, EMB))]
    nout = 2 if want_y else 1
    outs = pl.pallas_call(
        functools.partial(_post_body, want_y),
        grid=(GRID,), in_specs=in_specs,
        out_specs=[pl.BlockSpec((RB, EMB), lambda i: (i, 0))] * nout,
        out_shape=[jax.ShapeDtypeStruct((N, EMB), _f32)] * nout,
    )(s_raw, s_raw, right,
      fw, fb.reshape(1, EMB), pw, pb.reshape(1, EMB),
      o1w[:EMB], o1w[EMB:], o1b.reshape(1, EMB),
      o2w, o2b.reshape(1, EMB), zw, zb.reshape(1, EMB))
    return outs if want_y else (outs[0],)


def _ext_w(w):
    return jnp.pad(w, ((0, 0), (0, EXT - EMB)))


def _ext_b(b_or_none):
    b = jnp.zeros((EMB,), _f32) if b_or_none is None else b_or_none
    return jnp.concatenate([b, jnp.full((1,), 0.5, _f32),
                            jnp.zeros((EXT - EMB - 1,), _f32)])


def _ext_ew(ew):
    return jnp.concatenate([ew.reshape(EMB), jnp.zeros((EXT - EMB,), _f32)])


def _pad_edges(idx):
    pad = N + (jnp.arange(NE_P - NE, dtype=jnp.int32) % (NPAD - N))
    return jnp.concatenate([idx.astype(jnp.int32), pad])


def _edge_sc_body(a_hbm, b_hbm, src_hbm, dst_hbm, esc_hbm, ew_hbm, out_hbm,
                  ia0, ia1, ib0, ib1, io0, io1, es0, es1,
                  ab0, ab1, bb0, bb1, ewv, s_sp,
                  sa0, sa1, sb0, sb1, si0, si1):
    cid = lax.axis_index("c")
    sid = lax.axis_index("s")
    cofs = cid * N

    pltpu.sync_copy(ew_hbm.at[pl.ds(cid * HALF, HALF)], ewv)

    def zrow(k, c):
        for r in range(VPR):
            ab0[k, pl.ds(r * 16, 16)] = jnp.zeros((16,), _f32)
        return c
    lax.fori_loop(0, EK, zrow, 0)
    rbase = sid * RPT
    for t in range(RPT // EK):
        pltpu.sync_copy(ab0, s_sp.at[pl.ds(rbase + t * EK, EK), :])
    rem = RPT % EK
    if rem:
        pltpu.sync_copy(ab0.at[pl.ds(0, rem), :],
                        s_sp.at[pl.ds(rbase + RPT - rem, rem), :])
    plsc.subcore_barrier()

    ebase = sid * EPT
    slots = ((ia0, ib0, io0, es0, ab0, bb0, sa0, sb0, si0),
             (ia1, ib1, io1, es1, ab1, bb1, sa1, sb1, si1))

    def fire(j, slot):
        ia, ib, io, es, ab, bb, sa, sb, si = slot
        base = ebase + j * EK
        c1 = pltpu.async_copy(src_hbm.at[pl.ds(base, EK)], ia, si)
        c2 = pltpu.async_copy(dst_hbm.at[pl.ds(base, EK)], ib, si)
        c3 = pltpu.async_copy(esc_hbm.at[pl.ds(base, EK), :], es, si)
        c1.wait()
        c2.wait()
        c3.wait()
        for q in range(EK // 16):
            sl = pl.ds(q * 16, 16)
            ia[sl] = jnp.minimum(ia[sl], N - 1) + cofs
            io[sl] = jnp.minimum(ib[sl], N - 1) + cofs
        return (pltpu.async_copy(a_hbm.at[ia], ab, sa),
                pltpu.async_copy(b_hbm.at[io], bb, sb))

    def consume(slot, descs):
        ia, ib, io, es, ab, bb, sa, sb, si = slot
        descs[0].wait()
        descs[1].wait()

        @plsc.parallel_loop(0, EK, 1, unroll=4)
        def edge(k):
            ev = es[k, :]
            for r in range(VPR):
                sl = pl.ds(r * 16, 16)
                ab[k, sl] = jnp.maximum(
                    ab[k, sl] + bb[k, sl] + ev * ewv[sl], 0.0)
        pltpu.sync_copy(ab, s_sp.at[ib], add=True)

    def pair(jj, c):
        j0 = 2 * jj
        d0 = fire(j0, slots[0])
        d1 = fire(j0 + 1, slots[1])
        consume(slots[0], d0)
        consume(slots[1], d1)
        return c
    lax.fori_loop(0, NCH // 2, pair, 0)

    plsc.subcore_barrier()
    pltpu.sync_copy(s_sp.at[pl.ds(rbase, RPT), :],
                    out_hbm.at[cid, pl.ds(rbase, RPT), :])


def _edge_stage(a_tab, b_tab, src_p, dst_p, esc_p, ew_ext):
    fn = pl.kernel(
        _edge_sc_body,
        out_type=jax.ShapeDtypeStruct((2, NPAD, HALF), _f32),
        mesh=plsc.VectorSubcoreMesh(core_axis_name="c", subcore_axis_name="s",
                                    num_cores=2, num_subcores=NSUB),
        scratch_types=(
            [pltpu.VMEM((EK,), jnp.int32)] * 6
            + [pltpu.VMEM((EK, 16), _f32)] * 2
            + [pltpu.VMEM((EK, HALF), _f32)] * 4
            + [pltpu.VMEM((HALF,), _f32),
               pltpu.VMEM_SHARED((NPAD, HALF), _f32)]
            + [pltpu.SemaphoreType.DMA] * 6
        ),
        compiler_params=pltpu.CompilerParams(use_tc_tiling_on_sc=False),
    )
    return fn(a_tab, b_tab, src_p, dst_p, esc_p, ew_ext)


def kernel(constraint_features, edge_indices, edge_features, variable_features,
           ce, ee, ve, vc, cv, ov, oc):
    (vc_lw, vc_lb, vc_ew, vc_rw, vc_fw, vc_fb, vc_pw, vc_pb,
     vc_o1w, vc_o1b, vc_o2w, vc_o2b) = vc
    (cv_lw, cv_lb, cv_ew, cv_rw, cv_fw, cv_fb, cv_pw, cv_pb,
     cv_o1w, cv_o1b, cv_o2w, cv_o2b) = cv

    src_c = _pad_edges(edge_indices[0])
    src_v = _pad_edges(edge_indices[1])

    c1 = _embed(constraint_features, ce[0], ce[1], ce[2], ce[3])
    v1 = _embed(variable_features, ve[0], ve[1], ve[2], ve[3])
    (tb1,) = _split_tables(c1, [(_ext_w(vc_rw), _ext_b(None))])
    ta1, tb2 = _split_tables(v1, [(_ext_w(vc_lw), _ext_b(vc_lb)),
                                  (_ext_w(cv_rw), _ext_b(None))])
    esc = _edge_scalars(edge_features, ee[0], ee[1])
    esc_p = jnp.broadcast_to(
        jnp.concatenate([esc, jnp.zeros((NE_P - NE,), _f32)])[:, None],
        (NE_P, 16))

    s1 = _edge_stage(ta1, tb1, src_v, src_c, esc_p, _ext_ew(vc_ew))
    z_c, c2 = _post(s1, c1, vc, oc[0], oc[1], want_y=True)
    (ta2,) = _split_tables(c2, [(_ext_w(cv_lw), _ext_b(cv_lb))])

    s2 = _edge_stage(ta2, tb2, src_c, src_v, esc_p, _ext_ew(cv_ew))
    (z_v,) = _post(s2, v1, cv, ov[0], ov[1], want_y=False)

    return (z_v, z_c)
